# baseline (speedup 1.0000x reference)
import jax
import jax.numpy as jnp
from jax import lax
from jax.experimental import pallas as pl
from jax.experimental.pallas import tpu as pltpu
from jax.experimental.pallas import tpu_sc as plsc

NNODE = 10000
NEDGE = 320000
NEG = 0.2
NW = 32
EPW = NEDGE // NW
RPT = NNODE // 16
CB1 = 16
NCH1 = EPW // CB1
CB2 = 80
NCH2 = EPW // CB2


def _lr(x):
    return jnp.where(x >= 0, x, NEG * x)


def _allsum16(v):
    io = lax.iota(jnp.int32, 16)
    dn = lax.GatherDimensionNumbers(
        offset_dims=(), collapsed_slice_dims=(0,), start_index_map=(0,))
    for sh in (8, 4, 2, 1):
        p = lax.gather(v, (io ^ sh)[:, None], dn, (1,),
                       mode=lax.GatherScatterMode.PROMISE_IN_BOUNDS)
        v = v + p
    return v


def _mm_body(x_ref, w_ref, o_ref):
    o_ref[...] = jnp.dot(x_ref[...], w_ref[...], preferred_element_type=jnp.float32)


def _matmul(x, w, blk_rows):
    n = x.shape[0]
    return pl.pallas_call(
        _mm_body,
        grid=(n // blk_rows,),
        in_specs=[
            pl.BlockSpec((blk_rows, x.shape[1]), lambda i: (i, 0)),
            pl.BlockSpec(w.shape, lambda i: (0, 0)),
        ],
        out_specs=pl.BlockSpec((blk_rows, w.shape[1]), lambda i: (i, 0)),
        out_shape=jax.ShapeDtypeStruct((n, w.shape[1]), jnp.float32),
    )(x, w)


def _edge_proj_body(ea_ref, w1_ref, w2_ref, o1_ref, o2_ref):
    a = ea_ref[...]
    o1_ref[...] = jnp.dot(a, w1_ref[...], preferred_element_type=jnp.float32)
    o2_ref[...] = jnp.dot(a, w2_ref[...], preferred_element_type=jnp.float32)


def _edge_proj(ea, we1, we23):
    blk = 4000
    return pl.pallas_call(
        _edge_proj_body,
        grid=(NEDGE // blk,),
        in_specs=[
            pl.BlockSpec((blk, 16), lambda i: (i, 0)),
            pl.BlockSpec((16, 128), lambda i: (0, 0)),
            pl.BlockSpec((16, 32), lambda i: (0, 0)),
        ],
        out_specs=[
            pl.BlockSpec((blk, 128), lambda i: (i, 0)),
            pl.BlockSpec((blk, 32), lambda i: (i, 0)),
        ],
        out_shape=[
            jax.ShapeDtypeStruct((NEDGE, 128), jnp.float32),
            jax.ShapeDtypeStruct((NEDGE, 32), jnp.float32),
        ],
    )(ea, we1, we23)


def _combine1_body(p0_ref, p1_ref, e0_ref, e1_ref, m0_ref, m1_ref, c0_ref, c1_ref,
                   xl_ref, we1_ref, att1_ref, b1_ref,
                   w23_ref, we23_ref, att23_ref, xf_ref, es_ref):
    accv = p0_ref[...] + p1_ref[...]
    eesum = e0_ref[...][:, 0:1] + e1_ref[...][:, 0:1]
    cnt = c0_ref[...][:, 0:1] + c1_ref[...][:, 0:1]
    easum = m0_ref[...] + m1_ref[...]
    mean = easum / jnp.maximum(cnt, 1.0)
    xl = xl_ref[...]
    mw1 = jnp.dot(mean, we1_ref[...], preferred_element_type=jnp.float32)
    t1 = _lr(2.0 * xl + mw1)
    e1s = jnp.sum(t1 * att1_ref[...], axis=1, keepdims=True)
    ee1 = jnp.exp(e1s)
    latent = (accv + ee1 * xl) / (eesum + ee1) + b1_ref[...]
    xf = jnp.dot(latent, w23_ref[...], preferred_element_type=jnp.float32)
    mw23 = jnp.dot(mean, we23_ref[...], preferred_element_type=jnp.float32)
    t = _lr(2.0 * xf + mw23)
    w = att23_ref[...]
    e3s = jnp.sum(t[:, :16] * w[:, :16], axis=1, keepdims=True)
    e2s = jnp.sum(t[:, 16:] * w[:, 16:], axis=1, keepdims=True)
    xf_ref[...] = xf
    es_ref[...] = jnp.concatenate(
        [jnp.exp(e2s), jnp.exp(e3s), jnp.zeros((xf.shape[0], 6), jnp.float32)], axis=1)


def _combine1(p0, p1, e0, e1, m0, m1, c0, c1, xl1, we1, att1, b1, w23, we23, att23):
    blk = 400
    return pl.pallas_call(
        _combine1_body,
        grid=(NNODE // blk,),
        in_specs=[
            pl.BlockSpec((blk, 128), lambda i: (i, 0)),
            pl.BlockSpec((blk, 128), lambda i: (i, 0)),
            pl.BlockSpec((blk, 16), lambda i: (i, 0)),
            pl.BlockSpec((blk, 16), lambda i: (i, 0)),
            pl.BlockSpec((blk, 16), lambda i: (i, 0)),
            pl.BlockSpec((blk, 16), lambda i: (i, 0)),
            pl.BlockSpec((blk, 16), lambda i: (i, 0)),
            pl.BlockSpec((blk, 16), lambda i: (i, 0)),
            pl.BlockSpec((blk, 128), lambda i: (i, 0)),
            pl.BlockSpec((16, 128), lambda i: (0, 0)),
            pl.BlockSpec((1, 128), lambda i: (0, 0)),
            pl.BlockSpec((1, 128), lambda i: (0, 0)),
            pl.BlockSpec((128, 32), lambda i: (0, 0)),
            pl.BlockSpec((16, 32), lambda i: (0, 0)),
            pl.BlockSpec((1, 32), lambda i: (0, 0)),
        ],
        out_specs=[
            pl.BlockSpec((blk, 32), lambda i: (i, 0)),
            pl.BlockSpec((blk, 8), lambda i: (i, 0)),
        ],
        out_shape=[
            jax.ShapeDtypeStruct((NNODE, 32), jnp.float32),
            jax.ShapeDtypeStruct((NNODE, 8), jnp.float32),
        ],
    )(p0, p1, e0, e1, m0, m1, c0, c1, xl1, we1, att1, b1, w23, we23, att23)


def _combine2_body(q0_ref, q1_ref, xf_ref, es_ref, b2_ref, b3_ref, nl_ref, al_ref):
    s = q0_ref[...] + q1_ref[...]
    acc3 = s[:, :16]
    acc2 = s[:, 16:17]
    d2 = s[:, 17:18]
    d3 = s[:, 18:19]
    es = es_ref[...]
    ee2 = es[:, 0:1]
    ee3 = es[:, 1:2]
    xf = xf_ref[...]
    nl_ref[...] = (acc2 + ee2 * xf[:, 16:17]) / (d2 + ee2) + b2_ref[...]
    al_ref[...] = (acc3 + ee3 * xf[:, :16]) / (d3 + ee3) + b3_ref[...]


def _combine2(q0, q1, xf, es, b2, b3):
    blk = 400
    return pl.pallas_call(
        _combine2_body,
        grid=(NNODE // blk,),
        in_specs=[
            pl.BlockSpec((blk, 32), lambda i: (i, 0)),
            pl.BlockSpec((blk, 32), lambda i: (i, 0)),
            pl.BlockSpec((blk, 32), lambda i: (i, 0)),
            pl.BlockSpec((blk, 8), lambda i: (i, 0)),
            pl.BlockSpec((1, 1), lambda i: (0, 0)),
            pl.BlockSpec((1, 16), lambda i: (0, 0)),
        ],
        out_specs=[
            pl.BlockSpec((blk, 1), lambda i: (i, 0)),
            pl.BlockSpec((blk, 16), lambda i: (i, 0)),
        ],
        out_shape=[
            jax.ShapeDtypeStruct((NNODE, 1), jnp.float32),
            jax.ShapeDtypeStruct((NNODE, 16), jnp.float32),
        ],
    )(q0, q1, xf, es, b2, b3)


def _sc_mesh():
    return plsc.VectorSubcoreMesh(
        core_axis_name="c", subcore_axis_name="s", num_cores=2, num_subcores=16)


def _sc0_body(ei_hbm, ea_hbm, z_hbm, mout_hbm, cout_hbm,
              accm, accc, dstv, eav, ones, sem_d, sem_e, sem_sm, sem_sc):
    cid = lax.axis_index("c")
    sid = lax.axis_index("s")
    ebase = _wid_of(cid, sid) * EPW
    r0 = sid * RPT

    pltpu.sync_copy(z_hbm.at[pl.ds(r0, RPT), pl.ds(0, 16)],
                    accm.at[pl.ds(r0, RPT), :])
    pltpu.sync_copy(z_hbm.at[pl.ds(r0, RPT), pl.ds(16, 16)],
                    accc.at[pl.ds(r0, RPT), :])

    @pl.loop(0, CB2)
    def _init(i):
        ones[i, :] = jnp.where(lax.iota(jnp.int32, 16) == 0, 1.0, 0.0)

    plsc.subcore_barrier()

    def d_dst(ch, s):
        return pltpu.make_async_copy(
            ei_hbm.at[1, pl.ds(ebase + ch * CB2, CB2)], dstv.at[s], sem_d.at[s])

    def d_ea(ch, s):
        return pltpu.make_async_copy(
            ea_hbm.at[pl.ds(ebase + ch * CB2, CB2), :], eav.at[s], sem_e.at[s])

    def d_sm(s, b):
        return pltpu.make_async_copy(eav.at[s], accm.at[dstv.at[s]], sem_sm.at[b])

    def d_sc(s, b):
        return pltpu.make_async_copy(ones, accc.at[dstv.at[s]], sem_sc.at[b])

    d_dst(0, 0).start()
    d_ea(0, 0).start()
    d_dst(1, 1).start()
    d_ea(1, 1).start()

    @pl.loop(0, NCH2)
    def _chunk(ch):
        s6 = lax.rem(ch, 6)
        for ss in range(6):
            @pl.when(s6 == ss)
            def _():
                bb = ss & 1
                s2 = (ss + 2) % 6

                @pl.when(ch + 2 < NCH2)
                def _():
                    d_dst(ch + 2, s2).start()
                    d_ea(ch + 2, s2).start()

                @pl.when(ch >= 2)
                def _():
                    d_sm((ss + 4) % 6, bb).wait()
                    d_sc((ss + 4) % 6, bb).wait()

                d_dst(ch, ss).wait()
                d_ea(ch, ss).wait()
                d_sm(ss, bb).start(add=True)
                d_sc(ss, bb).start(add=True)

    for chl in (NCH2 - 2, NCH2 - 1):
        d_sm(chl % 6, chl % 2).wait()
        d_sc(chl % 6, chl % 2).wait()

    plsc.subcore_barrier()
    pltpu.sync_copy(accm.at[pl.ds(r0, RPT), :], mout_hbm.at[cid, pl.ds(r0, RPT), :])
    pltpu.sync_copy(accc.at[pl.ds(r0, RPT), :], cout_hbm.at[cid, pl.ds(r0, RPT), :])


def _wid_of(cid, sid):
    return cid * 16 + sid


def _sc0_call(edge_index, edge_attr, z32):
    f = pl.kernel(
        _sc0_body,
        out_type=(jax.ShapeDtypeStruct((2, NNODE, 16), jnp.float32),
                  jax.ShapeDtypeStruct((2, NNODE, 16), jnp.float32)),
        mesh=_sc_mesh(),
        compiler_params=pltpu.CompilerParams(use_tc_tiling_on_sc=False),
        scratch_types=[
            pltpu.VMEM_SHARED((NNODE, 16), jnp.float32),
            pltpu.VMEM_SHARED((NNODE, 16), jnp.float32),
            pltpu.VMEM((6, CB2), jnp.int32),
            pltpu.VMEM((6, CB2, 16), jnp.float32),
            pltpu.VMEM((CB2, 16), jnp.float32),
            pltpu.SemaphoreType.DMA((6,)),
            pltpu.SemaphoreType.DMA((6,)),
            pltpu.SemaphoreType.DMA((2,)),
            pltpu.SemaphoreType.DMA((2,)),
        ],
    )
    return f(edge_index, edge_attr, z32)


def _sc1_body(xl1_hbm, g1_hbm, ei_hbm, att1_hbm, z128_hbm, z16_hbm,
              pout_hbm, eout_hbm,
              accp, acce, sd, comb, xdv, g1v, meta, att1v,
              sem_sd, sem_xs, sem_xd, sem_g, sem_scm, sem_sce):
    cid = lax.axis_index("c")
    sid = lax.axis_index("s")
    ebase = _wid_of(cid, sid) * EPW
    r0 = sid * RPT

    pltpu.sync_copy(z128_hbm.at[pl.ds(r0, RPT), :], accp.at[pl.ds(r0, RPT), :])
    pltpu.sync_copy(z16_hbm.at[pl.ds(r0, RPT), :], acce.at[pl.ds(r0, RPT), :])
    pltpu.sync_copy(att1_hbm, att1v)
    plsc.subcore_barrier()

    attc = [att1v[pl.ds(16 * k, 16)] for k in range(8)]
    io = lax.iota(jnp.int32, 16)
    l0 = jnp.where(io == 0, 1.0, 0.0).astype(jnp.float32)

    def d_sd(ch, s):
        return pltpu.make_async_copy(
            ei_hbm.at[:, pl.ds(ebase + ch * CB1, CB1)], sd.at[s], sem_sd.at[s])

    def d_xs(ch, s):
        return pltpu.make_async_copy(
            xl1_hbm.at[sd.at[s].at[0]], comb.at[s], sem_xs.at[s % 3])

    def d_xd(ch, s):
        return pltpu.make_async_copy(
            xl1_hbm.at[sd.at[s].at[1]], xdv.at[s % 3], sem_xd.at[s % 3])

    def d_g(ch, s):
        return pltpu.make_async_copy(
            g1_hbm.at[pl.ds(ebase + ch * CB1, CB1), :], g1v.at[s % 3], sem_g.at[s % 3])

    def d_scm(s):
        return pltpu.make_async_copy(
            comb.at[s], accp.at[sd.at[s].at[1]], sem_scm.at[s & 1])

    def d_sce(s):
        return pltpu.make_async_copy(
            meta.at[s & 1], acce.at[sd.at[s].at[1]], sem_sce.at[s & 1])

    pltpu.sync_copy(ei_hbm.at[:, pl.ds(ebase, CB1)], sd.at[0])
    pltpu.sync_copy(ei_hbm.at[:, pl.ds(ebase + CB1, CB1)], sd.at[1])
    d_xs(0, 0).start()
    d_xd(0, 0).start()
    d_g(0, 0).start()
    d_xs(1, 1).start()
    d_xd(1, 1).start()
    d_g(1, 1).start()
    d_sd(2, 2).start()

    @pl.loop(0, NCH1)
    def _chunk(ch):
        s6 = lax.rem(ch, 6)
        for ss in range(6):
            @pl.when(s6 == ss)
            def _():
                bb = ss & 1
                b3 = ss % 3
                s2 = (ss + 2) % 6
                s3 = (ss + 3) % 6

                @pl.when(ch + 2 < NCH1)
                def _():
                    d_sd(ch + 2, s2).wait()

                @pl.when(ch >= 2)
                def _():
                    d_scm((ss + 4) % 6).wait()
                    d_sce((ss + 4) % 6).wait()

                @pl.when(ch + 2 < NCH1)
                def _():
                    d_xs(ch + 2, s2).start()
                    d_xd(ch + 2, s2).start()
                    d_g(ch + 2, s2).start()

                @pl.when(ch + 3 < NCH1)
                def _():
                    d_sd(ch + 3, s3).start()

                d_xs(ch, ss).wait()
                d_xd(ch, ss).wait()
                d_g(ch, ss).wait()

                @pl.loop(0, CB1, unroll=4)
                def _edge(i):
                    xs = []
                    terms = []
                    for k in range(8):
                        a = comb[ss, i, pl.ds(16 * k, 16)]
                        m = (a + xdv[b3, i, pl.ds(16 * k, 16)]
                             + g1v[b3, i, pl.ds(16 * k, 16)])
                        terms.append(_lr(m) * attc[k])
                        xs.append(a)
                    t01 = terms[0] + terms[1]
                    t23 = terms[2] + terms[3]
                    t45 = terms[4] + terms[5]
                    t67 = terms[6] + terms[7]
                    acc = (t01 + t23) + (t45 + t67)
                    ee = jnp.exp(_allsum16(acc))
                    for k in range(8):
                        comb[ss, i, pl.ds(16 * k, 16)] = xs[k] * ee
                    meta[bb, i, :] = ee * l0

                d_scm(ss).start(add=True)
                d_sce(ss).start(add=True)

    for chl in (NCH1 - 2, NCH1 - 1):
        d_scm(chl % 6).wait()
        d_sce(chl % 6).wait()

    plsc.subcore_barrier()
    pltpu.sync_copy(accp.at[pl.ds(r0, RPT), :], pout_hbm.at[cid, pl.ds(r0, RPT), :])
    pltpu.sync_copy(acce.at[pl.ds(r0, RPT), :], eout_hbm.at[cid, pl.ds(r0, RPT), :])


def _sc1_call(xl1, g1, edge_index, att1, z128, z16):
    f = pl.kernel(
        _sc1_body,
        out_type=(jax.ShapeDtypeStruct((2, NNODE, 128), jnp.float32),
                  jax.ShapeDtypeStruct((2, NNODE, 16), jnp.float32)),
        mesh=_sc_mesh(),
        compiler_params=pltpu.CompilerParams(use_tc_tiling_on_sc=False),
        scratch_types=[
            pltpu.VMEM_SHARED((NNODE, 128), jnp.float32),
            pltpu.VMEM_SHARED((NNODE, 16), jnp.float32),
            pltpu.VMEM((6, 2, CB1), jnp.int32),
            pltpu.VMEM((6, CB1, 128), jnp.float32),
            pltpu.VMEM((3, CB1, 128), jnp.float32),
            pltpu.VMEM((3, CB1, 128), jnp.float32),
            pltpu.VMEM((2, CB1, 16), jnp.float32),
            pltpu.VMEM((128,), jnp.float32),
            pltpu.SemaphoreType.DMA((6,)),
            pltpu.SemaphoreType.DMA((3,)),
            pltpu.SemaphoreType.DMA((3,)),
            pltpu.SemaphoreType.DMA((3,)),
            pltpu.SemaphoreType.DMA((2,)),
            pltpu.SemaphoreType.DMA((2,)),
        ],
    )
    return f(xl1, g1, edge_index, att1, z128, z16)


def _sc23_body(xf_hbm, g23_hbm, ei_hbm, att23_hbm, z32_hbm, out_hbm,
               acc_sh, sd, comb, xdv, g23v, att23v,
               sem_sd, sem_xs, sem_xd, sem_g, sem_sc):
    cid = lax.axis_index("c")
    sid = lax.axis_index("s")
    ebase = _wid_of(cid, sid) * EPW
    r0 = sid * RPT

    pltpu.sync_copy(z32_hbm.at[pl.ds(r0, RPT), :], acc_sh.at[pl.ds(r0, RPT), :])
    pltpu.sync_copy(att23_hbm, att23v)
    plsc.subcore_barrier()

    att3 = att23v[pl.ds(0, 16)]
    att2h = att23v[pl.ds(16, 16)]
    io = lax.iota(jnp.int32, 16)
    l0 = jnp.where(io == 0, 1.0, 0.0).astype(jnp.float32)
    l1 = jnp.where(io == 1, 1.0, 0.0).astype(jnp.float32)
    l2 = jnp.where(io == 2, 1.0, 0.0).astype(jnp.float32)

    def d_sd(ch, s):
        return pltpu.make_async_copy(
            ei_hbm.at[:, pl.ds(ebase + ch * CB2, CB2)], sd.at[s], sem_sd.at[s])

    def d_xs(ch, s):
        return pltpu.make_async_copy(
            xf_hbm.at[sd.at[s].at[0]], comb.at[s], sem_xs.at[s % 3])

    def d_xd(ch, s):
        return pltpu.make_async_copy(
            xf_hbm.at[sd.at[s].at[1]], xdv.at[s % 3], sem_xd.at[s % 3])

    def d_g(ch, s):
        return pltpu.make_async_copy(
            g23_hbm.at[pl.ds(ebase + ch * CB2, CB2), :], g23v.at[s % 3],
            sem_g.at[s % 3])

    def d_sc(s):
        return pltpu.make_async_copy(
            comb.at[s], acc_sh.at[sd.at[s].at[1]], sem_sc.at[s & 1])

    pltpu.sync_copy(ei_hbm.at[:, pl.ds(ebase, CB2)], sd.at[0])
    pltpu.sync_copy(ei_hbm.at[:, pl.ds(ebase + CB2, CB2)], sd.at[1])
    d_xs(0, 0).start()
    d_xd(0, 0).start()
    d_g(0, 0).start()
    d_xs(1, 1).start()
    d_xd(1, 1).start()
    d_g(1, 1).start()
    d_sd(2, 2).start()

    @pl.loop(0, NCH2)
    def _chunk(ch):
        s6 = lax.rem(ch, 6)
        for ss in range(6):
            @pl.when(s6 == ss)
            def _():
                b3 = ss % 3
                s2 = (ss + 2) % 6
                s3 = (ss + 3) % 6

                @pl.when(ch + 2 < NCH2)
                def _():
                    d_sd(ch + 2, s2).wait()

                @pl.when(ch >= 2)
                def _():
                    d_sc((ss + 4) % 6).wait()

                @pl.when(ch + 2 < NCH2)
                def _():
                    d_xs(ch + 2, s2).start()
                    d_xd(ch + 2, s2).start()
                    d_g(ch + 2, s2).start()

                @pl.when(ch + 3 < NCH2)
                def _():
                    d_sd(ch + 3, s3).start()

                d_xs(ch, ss).wait()
                d_xd(ch, ss).wait()
                d_g(ch, ss).wait()

                @pl.loop(0, CB2, unroll=4)
                def _edge(i):
                    xs_lo = comb[ss, i, pl.ds(0, 16)]
                    xs_hi = comb[ss, i, pl.ds(16, 16)]
                    m3 = (xs_lo + xdv[b3, i, pl.ds(0, 16)]
                          + g23v[b3, i, pl.ds(0, 16)])
                    v2 = (xs_hi + xdv[b3, i, pl.ds(16, 16)]
                          + g23v[b3, i, pl.ds(16, 16)])
                    ee3 = jnp.exp(_allsum16(_lr(m3) * att3))
                    ee2 = jnp.exp(_allsum16(_lr(v2) * att2h))
                    comb[ss, i, pl.ds(0, 16)] = xs_lo * ee3
                    comb[ss, i, pl.ds(16, 16)] = (
                        ee2 * (xs_hi * l0 + l1) + ee3 * l2)

                d_sc(ss).start(add=True)

    for chl in (NCH2 - 2, NCH2 - 1):
        d_sc(chl % 6).wait()

    plsc.subcore_barrier()
    pltpu.sync_copy(acc_sh.at[pl.ds(r0, RPT), :], out_hbm.at[cid, pl.ds(r0, RPT), :])


def _sc23_call(xf, g23, edge_index, att23, z32):
    f = pl.kernel(
        _sc23_body,
        out_type=jax.ShapeDtypeStruct((2, NNODE, 32), jnp.float32),
        mesh=_sc_mesh(),
        compiler_params=pltpu.CompilerParams(use_tc_tiling_on_sc=False),
        scratch_types=[
            pltpu.VMEM_SHARED((NNODE, 32), jnp.float32),
            pltpu.VMEM((6, 2, CB2), jnp.int32),
            pltpu.VMEM((6, CB2, 32), jnp.float32),
            pltpu.VMEM((3, CB2, 32), jnp.float32),
            pltpu.VMEM((3, CB2, 32), jnp.float32),
            pltpu.VMEM((32,), jnp.float32),
            pltpu.SemaphoreType.DMA((6,)),
            pltpu.SemaphoreType.DMA((3,)),
            pltpu.SemaphoreType.DMA((3,)),
            pltpu.SemaphoreType.DMA((3,)),
            pltpu.SemaphoreType.DMA((2,)),
        ],
    )
    return f(xf, g23, edge_index, att23, z32)


def kernel(x, edge_index, edge_attr, W1, att1, We1, b1, W2, att2, We2, b2,
           W3, att3, We3, b3):
    f32 = jnp.float32

    we23 = jnp.concatenate([We3, We2, jnp.zeros((16, 15), f32)], axis=1)

    xl1 = _matmul(x, W1, 400)
    g1, g23 = _edge_proj(edge_attr, We1, we23)

    z32 = jnp.zeros((NNODE, 32), f32)
    z128 = jnp.zeros((NNODE, 128), f32)
    z16 = jnp.zeros((NNODE, 16), f32)

    mp, cp = _sc0_call(edge_index, edge_attr, z32)

    p, e = _sc1_call(xl1, g1, edge_index, att1, z128, z16)

    att23 = jnp.concatenate([att3, att2, jnp.zeros((15,), f32)]).reshape(1, 32)
    w23 = jnp.concatenate([W3, W2, jnp.zeros((128, 15), f32)], axis=1)
    xf, es = _combine1(p[0], p[1], e[0], e[1], mp[0], mp[1], cp[0], cp[1],
                       xl1, We1, att1.reshape(1, 128), b1.reshape(1, 128),
                       w23, we23, att23)

    q = _sc23_call(xf, g23, edge_index, att23.reshape(32), z32)

    nl, al = _combine2(q[0], q[1], xf, es, b2.reshape(1, 1), b3.reshape(1, 16))
    node_logits = nl[:, 0]
    action_logits = al

    node_sel = jax.random.categorical(jax.random.key(42), node_logits)
    node_lp = jax.nn.log_softmax(node_logits)[node_sel]
    alr = action_logits[node_sel, :]
    act_sel = jax.random.categorical(jax.random.key(43), alr)
    act_lp = jax.nn.log_softmax(alr)[act_sel]
    return (node_sel, act_sel, node_lp + act_lp)


# speedup vs baseline: 1.1396x; 1.1396x over previous
import jax
import jax.numpy as jnp
from jax import lax
from jax.experimental import pallas as pl
from jax.experimental.pallas import tpu as pltpu
from jax.experimental.pallas import tpu_sc as plsc

NNODE = 10000
NEDGE = 320000
NEG = 0.2
NW = 32
EPW = NEDGE // NW
RPT = NNODE // 16
CB1 = 40
NCH1 = EPW // CB1
CB2 = 80
NCH2 = EPW // CB2


def _lr(x):
    return jnp.where(x >= 0, x, NEG * x)


def _allsum16(v):
    io = lax.iota(jnp.int32, 16)
    dn = lax.GatherDimensionNumbers(
        offset_dims=(), collapsed_slice_dims=(0,), start_index_map=(0,))
    for sh in (8, 4, 2, 1):
        p = lax.gather(v, (io ^ sh)[:, None], dn, (1,),
                       mode=lax.GatherScatterMode.PROMISE_IN_BOUNDS)
        v = v + p
    return v


def _mm_ones_body(x_ref, w_ref, o_ref):
    blk = x_ref.shape[0]
    d = jnp.dot(x_ref[...], w_ref[...], preferred_element_type=jnp.float32)
    tail = jnp.where(lax.broadcasted_iota(jnp.int32, (blk, 16), 1) == 0, 1.0, 0.0)
    o_ref[...] = jnp.concatenate([d, tail], axis=1)


def _matmul_ones(x, w, blk_rows):
    n = x.shape[0]
    return pl.pallas_call(
        _mm_ones_body,
        grid=(n // blk_rows,),
        in_specs=[
            pl.BlockSpec((blk_rows, x.shape[1]), lambda i: (i, 0)),
            pl.BlockSpec(w.shape, lambda i: (0, 0)),
        ],
        out_specs=pl.BlockSpec((blk_rows, 144), lambda i: (i, 0)),
        out_shape=jax.ShapeDtypeStruct((n, 144), jnp.float32),
    )(x, w)


def _edge_proj_body(ea_ref, w1_ref, w2_ref, o1_ref, o2_ref):
    a = ea_ref[...]
    o1_ref[...] = jnp.dot(a, w1_ref[...], preferred_element_type=jnp.float32)
    o2_ref[...] = jnp.dot(a, w2_ref[...], preferred_element_type=jnp.float32)


def _edge_proj(ea, we1, we23):
    blk = 4000
    return pl.pallas_call(
        _edge_proj_body,
        grid=(NEDGE // blk,),
        in_specs=[
            pl.BlockSpec((blk, 16), lambda i: (i, 0)),
            pl.BlockSpec((16, 128), lambda i: (0, 0)),
            pl.BlockSpec((16, 32), lambda i: (0, 0)),
        ],
        out_specs=[
            pl.BlockSpec((blk, 128), lambda i: (i, 0)),
            pl.BlockSpec((blk, 32), lambda i: (i, 0)),
        ],
        out_shape=[
            jax.ShapeDtypeStruct((NEDGE, 128), jnp.float32),
            jax.ShapeDtypeStruct((NEDGE, 32), jnp.float32),
        ],
    )(ea, we1, we23)


def _combine1_body(p0_ref, p1_ref, m0_ref, m1_ref, c0_ref, c1_ref,
                   xl_ref, we1_ref, att1_ref, b1_ref,
                   w23_ref, we23_ref, att23_ref, xf_ref, es_ref):
    s = p0_ref[...] + p1_ref[...]
    accv = s[:, :128]
    eesum = s[:, 128:129]
    cnt = c0_ref[...][:, 0:1] + c1_ref[...][:, 0:1]
    easum = m0_ref[...] + m1_ref[...]
    mean = easum / jnp.maximum(cnt, 1.0)
    xl = xl_ref[...][:, :128]
    mw1 = jnp.dot(mean, we1_ref[...], preferred_element_type=jnp.float32)
    t1 = _lr(2.0 * xl + mw1)
    e1s = jnp.sum(t1 * att1_ref[...], axis=1, keepdims=True)
    ee1 = jnp.exp(e1s)
    latent = (accv + ee1 * xl) / (eesum + ee1) + b1_ref[...]
    xf = jnp.dot(latent, w23_ref[...], preferred_element_type=jnp.float32)
    mw23 = jnp.dot(mean, we23_ref[...], preferred_element_type=jnp.float32)
    t = _lr(2.0 * xf + mw23)
    w = att23_ref[...]
    e3s = jnp.sum(t[:, :16] * w[:, :16], axis=1, keepdims=True)
    e2s = jnp.sum(t[:, 16:] * w[:, 16:], axis=1, keepdims=True)
    xf_ref[...] = xf
    es_ref[...] = jnp.concatenate(
        [jnp.exp(e2s), jnp.exp(e3s), jnp.zeros((xf.shape[0], 6), jnp.float32)], axis=1)


def _combine1(p0, p1, m0, m1, c0, c1, xl1c, we1, att1, b1, w23, we23, att23):
    blk = 400
    return pl.pallas_call(
        _combine1_body,
        grid=(NNODE // blk,),
        in_specs=[
            pl.BlockSpec((blk, 144), lambda i: (i, 0)),
            pl.BlockSpec((blk, 144), lambda i: (i, 0)),
            pl.BlockSpec((blk, 16), lambda i: (i, 0)),
            pl.BlockSpec((blk, 16), lambda i: (i, 0)),
            pl.BlockSpec((blk, 16), lambda i: (i, 0)),
            pl.BlockSpec((blk, 16), lambda i: (i, 0)),
            pl.BlockSpec((blk, 144), lambda i: (i, 0)),
            pl.BlockSpec((16, 128), lambda i: (0, 0)),
            pl.BlockSpec((1, 128), lambda i: (0, 0)),
            pl.BlockSpec((1, 128), lambda i: (0, 0)),
            pl.BlockSpec((128, 32), lambda i: (0, 0)),
            pl.BlockSpec((16, 32), lambda i: (0, 0)),
            pl.BlockSpec((1, 32), lambda i: (0, 0)),
        ],
        out_specs=[
            pl.BlockSpec((blk, 32), lambda i: (i, 0)),
            pl.BlockSpec((blk, 8), lambda i: (i, 0)),
        ],
        out_shape=[
            jax.ShapeDtypeStruct((NNODE, 32), jnp.float32),
            jax.ShapeDtypeStruct((NNODE, 8), jnp.float32),
        ],
    )(p0, p1, m0, m1, c0, c1, xl1c, we1, att1, b1, w23, we23, att23)


def _combine2_body(q0_ref, q1_ref, xf_ref, es_ref, b2_ref, b3_ref, nl_ref, al_ref):
    s = q0_ref[...] + q1_ref[...]
    acc3 = s[:, :16]
    acc2 = s[:, 16:17]
    d2 = s[:, 17:18]
    d3 = s[:, 18:19]
    es = es_ref[...]
    ee2 = es[:, 0:1]
    ee3 = es[:, 1:2]
    xf = xf_ref[...]
    nl_ref[...] = (acc2 + ee2 * xf[:, 16:17]) / (d2 + ee2) + b2_ref[...]
    al_ref[...] = (acc3 + ee3 * xf[:, :16]) / (d3 + ee3) + b3_ref[...]


def _combine2(q0, q1, xf, es, b2, b3):
    blk = 400
    return pl.pallas_call(
        _combine2_body,
        grid=(NNODE // blk,),
        in_specs=[
            pl.BlockSpec((blk, 32), lambda i: (i, 0)),
            pl.BlockSpec((blk, 32), lambda i: (i, 0)),
            pl.BlockSpec((blk, 32), lambda i: (i, 0)),
            pl.BlockSpec((blk, 8), lambda i: (i, 0)),
            pl.BlockSpec((1, 1), lambda i: (0, 0)),
            pl.BlockSpec((1, 16), lambda i: (0, 0)),
        ],
        out_specs=[
            pl.BlockSpec((blk, 1), lambda i: (i, 0)),
            pl.BlockSpec((blk, 16), lambda i: (i, 0)),
        ],
        out_shape=[
            jax.ShapeDtypeStruct((NNODE, 1), jnp.float32),
            jax.ShapeDtypeStruct((NNODE, 16), jnp.float32),
        ],
    )(q0, q1, xf, es, b2, b3)


def _sc_mesh():
    return plsc.VectorSubcoreMesh(
        core_axis_name="c", subcore_axis_name="s", num_cores=2, num_subcores=16)


def _sc0_body(ei_hbm, ea_hbm, z_hbm, mout_hbm, cout_hbm,
              accm, accc, dstv, eav, ones, sem_d, sem_e, sem_sm, sem_sc):
    cid = lax.axis_index("c")
    sid = lax.axis_index("s")
    ebase = _wid_of(cid, sid) * EPW
    r0 = sid * RPT

    pltpu.sync_copy(z_hbm.at[pl.ds(r0, RPT), pl.ds(0, 16)],
                    accm.at[pl.ds(r0, RPT), :])
    pltpu.sync_copy(z_hbm.at[pl.ds(r0, RPT), pl.ds(16, 16)],
                    accc.at[pl.ds(r0, RPT), :])

    @pl.loop(0, CB2)
    def _init(i):
        ones[i, :] = jnp.where(lax.iota(jnp.int32, 16) == 0, 1.0, 0.0)

    plsc.subcore_barrier()

    def d_dst(ch, s):
        return pltpu.make_async_copy(
            ei_hbm.at[1, pl.ds(ebase + ch * CB2, CB2)], dstv.at[s], sem_d.at[s])

    def d_ea(ch, s):
        return pltpu.make_async_copy(
            ea_hbm.at[pl.ds(ebase + ch * CB2, CB2), :], eav.at[s], sem_e.at[s])

    def d_sm(s, b):
        return pltpu.make_async_copy(eav.at[s], accm.at[dstv.at[s]], sem_sm.at[b])

    def d_sc(s, b):
        return pltpu.make_async_copy(ones, accc.at[dstv.at[s]], sem_sc.at[b])

    d_dst(0, 0).start()
    d_ea(0, 0).start()
    d_dst(1, 1).start()
    d_ea(1, 1).start()

    @pl.loop(0, NCH2)
    def _chunk(ch):
        s6 = lax.rem(ch, 6)
        for ss in range(6):
            @pl.when(s6 == ss)
            def _():
                bb = ss & 1
                s2 = (ss + 2) % 6

                @pl.when(ch + 2 < NCH2)
                def _():
                    d_dst(ch + 2, s2).start()
                    d_ea(ch + 2, s2).start()

                @pl.when(ch >= 2)
                def _():
                    d_sm((ss + 4) % 6, bb).wait()
                    d_sc((ss + 4) % 6, bb).wait()

                d_dst(ch, ss).wait()
                d_ea(ch, ss).wait()
                d_sm(ss, bb).start(add=True)
                d_sc(ss, bb).start(add=True)

    for chl in (NCH2 - 2, NCH2 - 1):
        d_sm(chl % 6, chl % 2).wait()
        d_sc(chl % 6, chl % 2).wait()

    plsc.subcore_barrier()
    pltpu.sync_copy(accm.at[pl.ds(r0, RPT), :], mout_hbm.at[cid, pl.ds(r0, RPT), :])
    pltpu.sync_copy(accc.at[pl.ds(r0, RPT), :], cout_hbm.at[cid, pl.ds(r0, RPT), :])


def _wid_of(cid, sid):
    return cid * 16 + sid


def _sc0_call(edge_index, edge_attr, z32):
    f = pl.kernel(
        _sc0_body,
        out_type=(jax.ShapeDtypeStruct((2, NNODE, 16), jnp.float32),
                  jax.ShapeDtypeStruct((2, NNODE, 16), jnp.float32)),
        mesh=_sc_mesh(),
        compiler_params=pltpu.CompilerParams(use_tc_tiling_on_sc=False),
        scratch_types=[
            pltpu.VMEM_SHARED((NNODE, 16), jnp.float32),
            pltpu.VMEM_SHARED((NNODE, 16), jnp.float32),
            pltpu.VMEM((6, CB2), jnp.int32),
            pltpu.VMEM((6, CB2, 16), jnp.float32),
            pltpu.VMEM((CB2, 16), jnp.float32),
            pltpu.SemaphoreType.DMA((6,)),
            pltpu.SemaphoreType.DMA((6,)),
            pltpu.SemaphoreType.DMA((2,)),
            pltpu.SemaphoreType.DMA((2,)),
        ],
    )
    return f(edge_index, edge_attr, z32)


def _sc1_body(xl1c_hbm, g1_hbm, ei_hbm, att1_hbm, z144_hbm,
              pout_hbm,
              accp, sd, comb, xdv, g1v, att1v,
              sem_sd, sem_xs, sem_xd, sem_g, sem_scm):
    cid = lax.axis_index("c")
    sid = lax.axis_index("s")
    ebase = _wid_of(cid, sid) * EPW
    r0 = sid * RPT

    pltpu.sync_copy(z144_hbm.at[pl.ds(r0, RPT), :], accp.at[pl.ds(r0, RPT), :])
    pltpu.sync_copy(att1_hbm, att1v)
    plsc.subcore_barrier()

    attc = [att1v[pl.ds(16 * k, 16)] for k in range(8)]

    def d_sd(ch, s):
        return pltpu.make_async_copy(
            ei_hbm.at[:, pl.ds(ebase + ch * CB1, CB1)], sd.at[s], sem_sd.at[s])

    def d_xs(ch, s3, s4):
        return pltpu.make_async_copy(
            xl1c_hbm.at[sd.at[s4].at[0]], comb.at[s3], sem_xs.at[s3])

    def d_xd(ch, b2, s4):
        return pltpu.make_async_copy(
            xl1c_hbm.at[sd.at[s4].at[1]], xdv.at[b2], sem_xd.at[b2])

    def d_g(ch, b2):
        return pltpu.make_async_copy(
            g1_hbm.at[pl.ds(ebase + ch * CB1, CB1), :], g1v.at[b2], sem_g.at[b2])

    def d_scm(s3, s4, b2):
        return pltpu.make_async_copy(
            comb.at[s3], accp.at[sd.at[s4].at[1]], sem_scm.at[b2])

    pltpu.sync_copy(ei_hbm.at[:, pl.ds(ebase, CB1)], sd.at[0])
    d_xs(0, 0, 0).start()
    d_xd(0, 0, 0).start()
    d_g(0, 0).start()
    d_sd(1, 1).start()

    @pl.loop(0, NCH1)
    def _chunk(ch):
        s12 = lax.rem(ch, 12)
        for ss in range(12):
            @pl.when(s12 == ss)
            def _():
                b2 = ss % 2
                c3 = ss % 3
                c4 = ss % 4

                @pl.when(ch + 1 < NCH1)
                def _():
                    d_sd(ch + 1, (c4 + 1) % 4).wait()

                @pl.when(ch >= 2)
                def _():
                    d_scm((c3 + 1) % 3, (c4 + 2) % 4, b2).wait()

                @pl.when(ch + 1 < NCH1)
                def _():
                    d_xs(ch + 1, (c3 + 1) % 3, (c4 + 1) % 4).start()
                    d_xd(ch + 1, 1 - b2, (c4 + 1) % 4).start()
                    d_g(ch + 1, 1 - b2).start()

                @pl.when(ch + 2 < NCH1)
                def _():
                    d_sd(ch + 2, (c4 + 2) % 4).start()

                d_xs(ch, c3, c4).wait()
                d_xd(ch, b2, c4).wait()
                d_g(ch, b2).wait()

                @pl.loop(0, CB1)
                def _edge(i):
                    xs = []
                    terms = []
                    for k in range(8):
                        a = comb[c3, i, pl.ds(16 * k, 16)]
                        m = (a + xdv[b2, i, pl.ds(16 * k, 16)]
                             + g1v[b2, i, pl.ds(16 * k, 16)])
                        terms.append(_lr(m) * attc[k])
                        xs.append(a)
                    t01 = terms[0] + terms[1]
                    t23 = terms[2] + terms[3]
                    t45 = terms[4] + terms[5]
                    t67 = terms[6] + terms[7]
                    acc = (t01 + t23) + (t45 + t67)
                    ee = jnp.exp(_allsum16(acc))
                    for k in range(8):
                        comb[c3, i, pl.ds(16 * k, 16)] = xs[k] * ee
                    comb[c3, i, pl.ds(128, 16)] = comb[c3, i, pl.ds(128, 16)] * ee

                d_scm(c3, c4, b2).start(add=True)

    for chl in (NCH1 - 2, NCH1 - 1):
        d_scm(chl % 3, chl % 4, chl % 2).wait()

    plsc.subcore_barrier()
    pltpu.sync_copy(accp.at[pl.ds(r0, RPT), :], pout_hbm.at[cid, pl.ds(r0, RPT), :])


def _sc1_call(xl1c, g1, edge_index, att1, z144):
    f = pl.kernel(
        _sc1_body,
        out_type=jax.ShapeDtypeStruct((2, NNODE, 144), jnp.float32),
        mesh=_sc_mesh(),
        compiler_params=pltpu.CompilerParams(use_tc_tiling_on_sc=False),
        scratch_types=[
            pltpu.VMEM_SHARED((NNODE, 144), jnp.float32),
            pltpu.VMEM((4, 2, CB1), jnp.int32),
            pltpu.VMEM((3, CB1, 144), jnp.float32),
            pltpu.VMEM((2, CB1, 144), jnp.float32),
            pltpu.VMEM((2, CB1, 128), jnp.float32),
            pltpu.VMEM((128,), jnp.float32),
            pltpu.SemaphoreType.DMA((4,)),
            pltpu.SemaphoreType.DMA((3,)),
            pltpu.SemaphoreType.DMA((2,)),
            pltpu.SemaphoreType.DMA((2,)),
            pltpu.SemaphoreType.DMA((2,)),
        ],
    )
    return f(xl1c, g1, edge_index, att1, z144)


def _sc23_body(xf_hbm, g23_hbm, ei_hbm, att23_hbm, z32_hbm, out_hbm,
               acc_sh, sd, comb, xdv, g23v, att23v,
               sem_sd, sem_xs, sem_xd, sem_g, sem_sc):
    cid = lax.axis_index("c")
    sid = lax.axis_index("s")
    ebase = _wid_of(cid, sid) * EPW
    r0 = sid * RPT

    pltpu.sync_copy(z32_hbm.at[pl.ds(r0, RPT), :], acc_sh.at[pl.ds(r0, RPT), :])
    pltpu.sync_copy(att23_hbm, att23v)
    plsc.subcore_barrier()

    att3 = att23v[pl.ds(0, 16)]
    att2h = att23v[pl.ds(16, 16)]
    io = lax.iota(jnp.int32, 16)
    l0 = jnp.where(io == 0, 1.0, 0.0).astype(jnp.float32)
    l1 = jnp.where(io == 1, 1.0, 0.0).astype(jnp.float32)
    l2 = jnp.where(io == 2, 1.0, 0.0).astype(jnp.float32)

    def d_sd(ch, s):
        return pltpu.make_async_copy(
            ei_hbm.at[:, pl.ds(ebase + ch * CB2, CB2)], sd.at[s], sem_sd.at[s])

    def d_xs(ch, s):
        return pltpu.make_async_copy(
            xf_hbm.at[sd.at[s].at[0]], comb.at[s], sem_xs.at[s % 3])

    def d_xd(ch, s):
        return pltpu.make_async_copy(
            xf_hbm.at[sd.at[s].at[1]], xdv.at[s % 3], sem_xd.at[s % 3])

    def d_g(ch, s):
        return pltpu.make_async_copy(
            g23_hbm.at[pl.ds(ebase + ch * CB2, CB2), :], g23v.at[s % 3],
            sem_g.at[s % 3])

    def d_sc(s):
        return pltpu.make_async_copy(
            comb.at[s], acc_sh.at[sd.at[s].at[1]], sem_sc.at[s & 1])

    pltpu.sync_copy(ei_hbm.at[:, pl.ds(ebase, CB2)], sd.at[0])
    pltpu.sync_copy(ei_hbm.at[:, pl.ds(ebase + CB2, CB2)], sd.at[1])
    d_xs(0, 0).start()
    d_xd(0, 0).start()
    d_g(0, 0).start()
    d_xs(1, 1).start()
    d_xd(1, 1).start()
    d_g(1, 1).start()
    d_sd(2, 2).start()

    @pl.loop(0, NCH2)
    def _chunk(ch):
        s6 = lax.rem(ch, 6)
        for ss in range(6):
            @pl.when(s6 == ss)
            def _():
                b3 = ss % 3
                s2 = (ss + 2) % 6
                s3 = (ss + 3) % 6

                @pl.when(ch + 2 < NCH2)
                def _():
                    d_sd(ch + 2, s2).wait()

                @pl.when(ch >= 2)
                def _():
                    d_sc((ss + 4) % 6).wait()

                @pl.when(ch + 2 < NCH2)
                def _():
                    d_xs(ch + 2, s2).start()
                    d_xd(ch + 2, s2).start()
                    d_g(ch + 2, s2).start()

                @pl.when(ch + 3 < NCH2)
                def _():
                    d_sd(ch + 3, s3).start()

                d_xs(ch, ss).wait()
                d_xd(ch, ss).wait()
                d_g(ch, ss).wait()

                @pl.loop(0, CB2, unroll=4)
                def _edge(i):
                    xs_lo = comb[ss, i, pl.ds(0, 16)]
                    xs_hi = comb[ss, i, pl.ds(16, 16)]
                    m3 = (xs_lo + xdv[b3, i, pl.ds(0, 16)]
                          + g23v[b3, i, pl.ds(0, 16)])
                    v2 = (xs_hi + xdv[b3, i, pl.ds(16, 16)]
                          + g23v[b3, i, pl.ds(16, 16)])
                    ee3 = jnp.exp(_allsum16(_lr(m3) * att3))
                    ee2 = jnp.exp(_allsum16(_lr(v2) * att2h))
                    comb[ss, i, pl.ds(0, 16)] = xs_lo * ee3
                    comb[ss, i, pl.ds(16, 16)] = (
                        ee2 * (xs_hi * l0 + l1) + ee3 * l2)

                d_sc(ss).start(add=True)

    for chl in (NCH2 - 2, NCH2 - 1):
        d_sc(chl % 6).wait()

    plsc.subcore_barrier()
    pltpu.sync_copy(acc_sh.at[pl.ds(r0, RPT), :], out_hbm.at[cid, pl.ds(r0, RPT), :])


def _sc23_call(xf, g23, edge_index, att23, z32):
    f = pl.kernel(
        _sc23_body,
        out_type=jax.ShapeDtypeStruct((2, NNODE, 32), jnp.float32),
        mesh=_sc_mesh(),
        compiler_params=pltpu.CompilerParams(use_tc_tiling_on_sc=False),
        scratch_types=[
            pltpu.VMEM_SHARED((NNODE, 32), jnp.float32),
            pltpu.VMEM((6, 2, CB2), jnp.int32),
            pltpu.VMEM((6, CB2, 32), jnp.float32),
            pltpu.VMEM((3, CB2, 32), jnp.float32),
            pltpu.VMEM((3, CB2, 32), jnp.float32),
            pltpu.VMEM((32,), jnp.float32),
            pltpu.SemaphoreType.DMA((6,)),
            pltpu.SemaphoreType.DMA((3,)),
            pltpu.SemaphoreType.DMA((3,)),
            pltpu.SemaphoreType.DMA((3,)),
            pltpu.SemaphoreType.DMA((2,)),
        ],
    )
    return f(xf, g23, edge_index, att23, z32)


def kernel(x, edge_index, edge_attr, W1, att1, We1, b1, W2, att2, We2, b2,
           W3, att3, We3, b3):
    f32 = jnp.float32

    we23 = jnp.concatenate([We3, We2, jnp.zeros((16, 15), f32)], axis=1)

    xl1c = _matmul_ones(x, W1, 400)
    g1, g23 = _edge_proj(edge_attr, We1, we23)

    z32 = jnp.zeros((NNODE, 32), f32)
    z144 = jnp.zeros((NNODE, 144), f32)

    mp, cp = _sc0_call(edge_index, edge_attr, z32)

    p = _sc1_call(xl1c, g1, edge_index, att1, z144)

    att23 = jnp.concatenate([att3, att2, jnp.zeros((15,), f32)]).reshape(1, 32)
    w23 = jnp.concatenate([W3, W2, jnp.zeros((128, 15), f32)], axis=1)
    xf, es = _combine1(p[0], p[1], mp[0], mp[1], cp[0], cp[1],
                       xl1c, We1, att1.reshape(1, 128), b1.reshape(1, 128),
                       w23, we23, att23)

    q = _sc23_call(xf, g23, edge_index, att23.reshape(32), z32)

    nl, al = _combine2(q[0], q[1], xf, es, b2.reshape(1, 1), b3.reshape(1, 16))
    node_logits = nl[:, 0]
    action_logits = al

    node_sel = jax.random.categorical(jax.random.key(42), node_logits)
    node_lp = jax.nn.log_softmax(node_logits)[node_sel]
    alr = action_logits[node_sel, :]
    act_sel = jax.random.categorical(jax.random.key(43), alr)
    act_lp = jax.nn.log_softmax(alr)[act_sel]
    return (node_sel, act_sel, node_lp + act_lp)


# speedup vs baseline: 1.1461x; 1.0057x over previous
import jax
import jax.numpy as jnp
from jax import lax
from jax.experimental import pallas as pl
from jax.experimental.pallas import tpu as pltpu
from jax.experimental.pallas import tpu_sc as plsc

NNODE = 10000
NEDGE = 320000
NEG = 0.2
NW = 32
EPW = NEDGE // NW
RPT = NNODE // 16
CB1 = 40
NCH1 = EPW // CB1
CB2 = 80
NCH2 = EPW // CB2


def _lr(x):
    return jnp.where(x >= 0, x, NEG * x)


def _allsum16(v):
    io = lax.iota(jnp.int32, 16)
    dn = lax.GatherDimensionNumbers(
        offset_dims=(), collapsed_slice_dims=(0,), start_index_map=(0,))
    for sh in (8, 4, 2, 1):
        p = lax.gather(v, (io ^ sh)[:, None], dn, (1,),
                       mode=lax.GatherScatterMode.PROMISE_IN_BOUNDS)
        v = v + p
    return v


def _mm_ones_body(x_ref, w_ref, o_ref):
    blk = x_ref.shape[0]
    d = jnp.dot(x_ref[...], w_ref[...], preferred_element_type=jnp.float32)
    tail = jnp.where(lax.broadcasted_iota(jnp.int32, (blk, 16), 1) == 0, 1.0, 0.0)
    o_ref[...] = jnp.concatenate([d, tail], axis=1)


def _matmul_ones(x, w, blk_rows):
    n = x.shape[0]
    return pl.pallas_call(
        _mm_ones_body,
        grid=(n // blk_rows,),
        in_specs=[
            pl.BlockSpec((blk_rows, x.shape[1]), lambda i: (i, 0)),
            pl.BlockSpec(w.shape, lambda i: (0, 0)),
        ],
        out_specs=pl.BlockSpec((blk_rows, 144), lambda i: (i, 0)),
        out_shape=jax.ShapeDtypeStruct((n, 144), jnp.float32),
    )(x, w)


def _edge_proj_body(ea_ref, w1_ref, w2_ref, o1_ref, o2_ref):
    a = ea_ref[...]
    o1_ref[...] = jnp.dot(a, w1_ref[...], preferred_element_type=jnp.float32)
    o2_ref[...] = jnp.dot(a, w2_ref[...], preferred_element_type=jnp.float32)


def _edge_proj(ea, we1, we23):
    blk = 4000
    return pl.pallas_call(
        _edge_proj_body,
        grid=(NEDGE // blk,),
        in_specs=[
            pl.BlockSpec((blk, 16), lambda i: (i, 0)),
            pl.BlockSpec((16, 128), lambda i: (0, 0)),
            pl.BlockSpec((16, 32), lambda i: (0, 0)),
        ],
        out_specs=[
            pl.BlockSpec((blk, 128), lambda i: (i, 0)),
            pl.BlockSpec((blk, 32), lambda i: (i, 0)),
        ],
        out_shape=[
            jax.ShapeDtypeStruct((NEDGE, 128), jnp.float32),
            jax.ShapeDtypeStruct((NEDGE, 32), jnp.float32),
        ],
    )(ea, we1, we23)


def _combine1_body(p0_ref, p1_ref, m0_ref, m1_ref, c0_ref, c1_ref,
                   xl_ref, we1_ref, att1_ref, b1_ref,
                   w23_ref, we23_ref, att23_ref, xf_ref, es_ref):
    s = p0_ref[...] + p1_ref[...]
    accv = s[:, :128]
    eesum = s[:, 128:129]
    cnt = c0_ref[...][:, 0:1] + c1_ref[...][:, 0:1]
    easum = m0_ref[...] + m1_ref[...]
    mean = easum / jnp.maximum(cnt, 1.0)
    xl = xl_ref[...][:, :128]
    mw1 = jnp.dot(mean, we1_ref[...], preferred_element_type=jnp.float32)
    t1 = _lr(2.0 * xl + mw1)
    e1s = jnp.sum(t1 * att1_ref[...], axis=1, keepdims=True)
    ee1 = jnp.exp(e1s)
    latent = (accv + ee1 * xl) / (eesum + ee1) + b1_ref[...]
    xf = jnp.dot(latent, w23_ref[...], preferred_element_type=jnp.float32)
    mw23 = jnp.dot(mean, we23_ref[...], preferred_element_type=jnp.float32)
    t = _lr(2.0 * xf + mw23)
    w = att23_ref[...]
    e3s = jnp.sum(t[:, :16] * w[:, :16], axis=1, keepdims=True)
    e2s = jnp.sum(t[:, 16:] * w[:, 16:], axis=1, keepdims=True)
    xf_ref[...] = xf
    es_ref[...] = jnp.concatenate(
        [jnp.exp(e2s), jnp.exp(e3s), jnp.zeros((xf.shape[0], 6), jnp.float32)], axis=1)


def _combine1(p0, p1, m0, m1, c0, c1, xl1c, we1, att1, b1, w23, we23, att23):
    blk = 400
    return pl.pallas_call(
        _combine1_body,
        grid=(NNODE // blk,),
        in_specs=[
            pl.BlockSpec((blk, 144), lambda i: (i, 0)),
            pl.BlockSpec((blk, 144), lambda i: (i, 0)),
            pl.BlockSpec((blk, 16), lambda i: (i, 0)),
            pl.BlockSpec((blk, 16), lambda i: (i, 0)),
            pl.BlockSpec((blk, 16), lambda i: (i, 0)),
            pl.BlockSpec((blk, 16), lambda i: (i, 0)),
            pl.BlockSpec((blk, 144), lambda i: (i, 0)),
            pl.BlockSpec((16, 128), lambda i: (0, 0)),
            pl.BlockSpec((1, 128), lambda i: (0, 0)),
            pl.BlockSpec((1, 128), lambda i: (0, 0)),
            pl.BlockSpec((128, 32), lambda i: (0, 0)),
            pl.BlockSpec((16, 32), lambda i: (0, 0)),
            pl.BlockSpec((1, 32), lambda i: (0, 0)),
        ],
        out_specs=[
            pl.BlockSpec((blk, 32), lambda i: (i, 0)),
            pl.BlockSpec((blk, 8), lambda i: (i, 0)),
        ],
        out_shape=[
            jax.ShapeDtypeStruct((NNODE, 32), jnp.float32),
            jax.ShapeDtypeStruct((NNODE, 8), jnp.float32),
        ],
    )(p0, p1, m0, m1, c0, c1, xl1c, we1, att1, b1, w23, we23, att23)


def _combine2_body(q0_ref, q1_ref, xf_ref, es_ref, b2_ref, b3_ref, nl_ref, al_ref):
    s = q0_ref[...] + q1_ref[...]
    acc3 = s[:, :16]
    acc2 = s[:, 16:17]
    d2 = s[:, 17:18]
    d3 = s[:, 18:19]
    es = es_ref[...]
    ee2 = es[:, 0:1]
    ee3 = es[:, 1:2]
    xf = xf_ref[...]
    nl_ref[...] = (acc2 + ee2 * xf[:, 16:17]) / (d2 + ee2) + b2_ref[...]
    al_ref[...] = (acc3 + ee3 * xf[:, :16]) / (d3 + ee3) + b3_ref[...]


def _combine2(q0, q1, xf, es, b2, b3):
    blk = 400
    return pl.pallas_call(
        _combine2_body,
        grid=(NNODE // blk,),
        in_specs=[
            pl.BlockSpec((blk, 32), lambda i: (i, 0)),
            pl.BlockSpec((blk, 32), lambda i: (i, 0)),
            pl.BlockSpec((blk, 32), lambda i: (i, 0)),
            pl.BlockSpec((blk, 8), lambda i: (i, 0)),
            pl.BlockSpec((1, 1), lambda i: (0, 0)),
            pl.BlockSpec((1, 16), lambda i: (0, 0)),
        ],
        out_specs=[
            pl.BlockSpec((blk, 1), lambda i: (i, 0)),
            pl.BlockSpec((blk, 16), lambda i: (i, 0)),
        ],
        out_shape=[
            jax.ShapeDtypeStruct((NNODE, 1), jnp.float32),
            jax.ShapeDtypeStruct((NNODE, 16), jnp.float32),
        ],
    )(q0, q1, xf, es, b2, b3)


def _sc_mesh():
    return plsc.VectorSubcoreMesh(
        core_axis_name="c", subcore_axis_name="s", num_cores=2, num_subcores=16)


def _sc0_body(ei_hbm, ea_hbm, z_hbm, mout_hbm, cout_hbm,
              accm, accc, dstv, eav, ones, sem_d, sem_e, sem_sm, sem_sc):
    cid = lax.axis_index("c")
    sid = lax.axis_index("s")
    ebase = _wid_of(cid, sid) * EPW
    r0 = sid * RPT

    pltpu.sync_copy(z_hbm.at[pl.ds(r0, RPT), pl.ds(0, 16)],
                    accm.at[pl.ds(r0, RPT), :])
    pltpu.sync_copy(z_hbm.at[pl.ds(r0, RPT), pl.ds(16, 16)],
                    accc.at[pl.ds(r0, RPT), :])

    @pl.loop(0, CB2)
    def _init(i):
        ones[i, :] = jnp.where(lax.iota(jnp.int32, 16) == 0, 1.0, 0.0)

    plsc.subcore_barrier()

    def d_dst(ch, s):
        return pltpu.make_async_copy(
            ei_hbm.at[1, pl.ds(ebase + ch * CB2, CB2)], dstv.at[s], sem_d.at[s])

    def d_ea(ch, s):
        return pltpu.make_async_copy(
            ea_hbm.at[pl.ds(ebase + ch * CB2, CB2), :], eav.at[s], sem_e.at[s])

    def d_sm(s, b):
        return pltpu.make_async_copy(eav.at[s], accm.at[dstv.at[s]], sem_sm.at[b])

    def d_sc(s, b):
        return pltpu.make_async_copy(ones, accc.at[dstv.at[s]], sem_sc.at[b])

    d_dst(0, 0).start()
    d_ea(0, 0).start()
    d_dst(1, 1).start()
    d_ea(1, 1).start()

    @pl.loop(0, NCH2)
    def _chunk(ch):
        s6 = lax.rem(ch, 6)
        for ss in range(6):
            @pl.when(s6 == ss)
            def _():
                bb = ss & 1
                s2 = (ss + 2) % 6

                @pl.when(ch + 2 < NCH2)
                def _():
                    d_dst(ch + 2, s2).start()
                    d_ea(ch + 2, s2).start()

                @pl.when(ch >= 2)
                def _():
                    d_sm((ss + 4) % 6, bb).wait()
                    d_sc((ss + 4) % 6, bb).wait()

                d_dst(ch, ss).wait()
                d_ea(ch, ss).wait()
                d_sm(ss, bb).start(add=True)
                d_sc(ss, bb).start(add=True)

    for chl in (NCH2 - 2, NCH2 - 1):
        d_sm(chl % 6, chl % 2).wait()
        d_sc(chl % 6, chl % 2).wait()

    plsc.subcore_barrier()
    pltpu.sync_copy(accm.at[pl.ds(r0, RPT), :], mout_hbm.at[cid, pl.ds(r0, RPT), :])
    pltpu.sync_copy(accc.at[pl.ds(r0, RPT), :], cout_hbm.at[cid, pl.ds(r0, RPT), :])


def _wid_of(cid, sid):
    return cid * 16 + sid


def _sc0_call(edge_index, edge_attr, z32):
    f = pl.kernel(
        _sc0_body,
        out_type=(jax.ShapeDtypeStruct((2, NNODE, 16), jnp.float32),
                  jax.ShapeDtypeStruct((2, NNODE, 16), jnp.float32)),
        mesh=_sc_mesh(),
        compiler_params=pltpu.CompilerParams(use_tc_tiling_on_sc=False),
        scratch_types=[
            pltpu.VMEM_SHARED((NNODE, 16), jnp.float32),
            pltpu.VMEM_SHARED((NNODE, 16), jnp.float32),
            pltpu.VMEM((6, CB2), jnp.int32),
            pltpu.VMEM((6, CB2, 16), jnp.float32),
            pltpu.VMEM((CB2, 16), jnp.float32),
            pltpu.SemaphoreType.DMA((6,)),
            pltpu.SemaphoreType.DMA((6,)),
            pltpu.SemaphoreType.DMA((2,)),
            pltpu.SemaphoreType.DMA((2,)),
        ],
    )
    return f(edge_index, edge_attr, z32)


def _sc1_body(xl1c_hbm, g1_hbm, ei_hbm, att1_hbm, z144_hbm,
              pout_hbm,
              accp, sd, comb, xdv, g1v, att1v,
              sem_sd, sem_xs, sem_xd, sem_g, sem_scm):
    cid = lax.axis_index("c")
    sid = lax.axis_index("s")
    ebase = _wid_of(cid, sid) * EPW
    r0 = sid * RPT

    pltpu.sync_copy(z144_hbm.at[pl.ds(r0, RPT), :], accp.at[pl.ds(r0, RPT), :])
    pltpu.sync_copy(att1_hbm, att1v)
    plsc.subcore_barrier()

    attc = [att1v[pl.ds(16 * k, 16)] for k in range(8)]

    def d_sd(ch, s):
        return pltpu.make_async_copy(
            ei_hbm.at[:, pl.ds(ebase + ch * CB1, CB1)], sd.at[s], sem_sd.at[s])

    def d_xs(ch, s3, s4):
        return pltpu.make_async_copy(
            xl1c_hbm.at[sd.at[s4].at[0]], comb.at[s3], sem_xs.at[s3])

    def d_xd(ch, b2, s4):
        return pltpu.make_async_copy(
            xl1c_hbm.at[sd.at[s4].at[1]], xdv.at[b2], sem_xd.at[b2])

    def d_g(ch, b2):
        return pltpu.make_async_copy(
            g1_hbm.at[pl.ds(ebase + ch * CB1, CB1), :], g1v.at[b2], sem_g.at[b2])

    def d_scm(s3, s4, b2):
        return pltpu.make_async_copy(
            comb.at[s3], accp.at[sd.at[s4].at[1]], sem_scm.at[b2])

    pltpu.sync_copy(ei_hbm.at[:, pl.ds(ebase, CB1)], sd.at[0])
    d_xs(0, 0, 0).start()
    d_xd(0, 0, 0).start()
    d_g(0, 0).start()
    d_sd(1, 1).start()

    @pl.loop(0, NCH1)
    def _chunk(ch):
        s12 = lax.rem(ch, 12)
        for ss in range(12):
            @pl.when(s12 == ss)
            def _():
                b2 = ss % 2
                c3 = ss % 3
                c4 = ss % 4

                @pl.when(ch + 1 < NCH1)
                def _():
                    d_sd(ch + 1, (c4 + 1) % 4).wait()

                @pl.when(ch >= 2)
                def _():
                    d_scm((c3 + 1) % 3, (c4 + 2) % 4, b2).wait()

                @pl.when(ch + 1 < NCH1)
                def _():
                    d_xs(ch + 1, (c3 + 1) % 3, (c4 + 1) % 4).start()
                    d_xd(ch + 1, 1 - b2, (c4 + 1) % 4).start()
                    d_g(ch + 1, 1 - b2).start()

                @pl.when(ch + 2 < NCH1)
                def _():
                    d_sd(ch + 2, (c4 + 2) % 4).start()

                d_xs(ch, c3, c4).wait()
                d_xd(ch, b2, c4).wait()
                d_g(ch, b2).wait()

                @pl.loop(0, CB1)
                def _edge(i):
                    xs = []
                    terms = []
                    for k in range(8):
                        a = comb[c3, i, pl.ds(16 * k, 16)]
                        m = (a + xdv[b2, i, pl.ds(16 * k, 16)]
                             + g1v[b2, i, pl.ds(16 * k, 16)])
                        terms.append(_lr(m) * attc[k])
                        xs.append(a)
                    t01 = terms[0] + terms[1]
                    t23 = terms[2] + terms[3]
                    t45 = terms[4] + terms[5]
                    t67 = terms[6] + terms[7]
                    acc = (t01 + t23) + (t45 + t67)
                    ee = jnp.exp(_allsum16(acc))
                    for k in range(8):
                        comb[c3, i, pl.ds(16 * k, 16)] = xs[k] * ee
                    comb[c3, i, pl.ds(128, 16)] = comb[c3, i, pl.ds(128, 16)] * ee

                d_scm(c3, c4, b2).start(add=True)

    for chl in (NCH1 - 2, NCH1 - 1):
        d_scm(chl % 3, chl % 4, chl % 2).wait()

    plsc.subcore_barrier()
    pltpu.sync_copy(accp.at[pl.ds(r0, RPT), :], pout_hbm.at[cid, pl.ds(r0, RPT), :])


def _sc1_call(xl1c, g1, edge_index, att1, z144):
    f = pl.kernel(
        _sc1_body,
        out_type=jax.ShapeDtypeStruct((2, NNODE, 144), jnp.float32),
        mesh=_sc_mesh(),
        compiler_params=pltpu.CompilerParams(use_tc_tiling_on_sc=False),
        scratch_types=[
            pltpu.VMEM_SHARED((NNODE, 144), jnp.float32),
            pltpu.VMEM((4, 2, CB1), jnp.int32),
            pltpu.VMEM((3, CB1, 144), jnp.float32),
            pltpu.VMEM((2, CB1, 144), jnp.float32),
            pltpu.VMEM((2, CB1, 128), jnp.float32),
            pltpu.VMEM((128,), jnp.float32),
            pltpu.SemaphoreType.DMA((4,)),
            pltpu.SemaphoreType.DMA((3,)),
            pltpu.SemaphoreType.DMA((2,)),
            pltpu.SemaphoreType.DMA((2,)),
            pltpu.SemaphoreType.DMA((2,)),
        ],
    )
    return f(xl1c, g1, edge_index, att1, z144)


def _sc23_body(xf_hbm, g23_hbm, ei_hbm, att23_hbm, z32_hbm, out_hbm,
               acc_sh, sd, comb, xdv, g23v, att23v,
               sem_sd, sem_xs, sem_xd, sem_g, sem_sc):
    cid = lax.axis_index("c")
    sid = lax.axis_index("s")
    ebase = _wid_of(cid, sid) * EPW
    r0 = sid * RPT

    pltpu.sync_copy(z32_hbm.at[pl.ds(r0, RPT), :], acc_sh.at[pl.ds(r0, RPT), :])
    pltpu.sync_copy(att23_hbm, att23v)
    plsc.subcore_barrier()

    att3 = att23v[pl.ds(0, 16)]
    att2h = att23v[pl.ds(16, 16)]
    io = lax.iota(jnp.int32, 16)
    l0 = jnp.where(io == 0, 1.0, 0.0).astype(jnp.float32)
    l1 = jnp.where(io == 1, 1.0, 0.0).astype(jnp.float32)
    l2 = jnp.where(io == 2, 1.0, 0.0).astype(jnp.float32)

    def d_sd(ch, s):
        return pltpu.make_async_copy(
            ei_hbm.at[:, pl.ds(ebase + ch * CB2, CB2)], sd.at[s], sem_sd.at[s])

    def d_xs(ch, s3, s4):
        return pltpu.make_async_copy(
            xf_hbm.at[sd.at[s4].at[0]], comb.at[s3], sem_xs.at[s3])

    def d_xd(ch, b2, s4):
        return pltpu.make_async_copy(
            xf_hbm.at[sd.at[s4].at[1]], xdv.at[b2], sem_xd.at[b2])

    def d_g(ch, b2):
        return pltpu.make_async_copy(
            g23_hbm.at[pl.ds(ebase + ch * CB2, CB2), :], g23v.at[b2],
            sem_g.at[b2])

    def d_sc(s3, s4, b2):
        return pltpu.make_async_copy(
            comb.at[s3], acc_sh.at[sd.at[s4].at[1]], sem_sc.at[b2])

    pltpu.sync_copy(ei_hbm.at[:, pl.ds(ebase, CB2)], sd.at[0])
    d_xs(0, 0, 0).start()
    d_xd(0, 0, 0).start()
    d_g(0, 0).start()
    d_sd(1, 1).start()

    @pl.loop(0, NCH2)
    def _chunk(ch):
        s12 = lax.rem(ch, 12)
        for ss in range(12):
            @pl.when(s12 == ss)
            def _():
                b2 = ss % 2
                c3 = ss % 3
                c4 = ss % 4

                @pl.when(ch + 1 < NCH2)
                def _():
                    d_sd(ch + 1, (c4 + 1) % 4).wait()

                @pl.when(ch >= 2)
                def _():
                    d_sc((c3 + 1) % 3, (c4 + 2) % 4, b2).wait()

                @pl.when(ch + 1 < NCH2)
                def _():
                    d_xs(ch + 1, (c3 + 1) % 3, (c4 + 1) % 4).start()
                    d_xd(ch + 1, 1 - b2, (c4 + 1) % 4).start()
                    d_g(ch + 1, 1 - b2).start()

                @pl.when(ch + 2 < NCH2)
                def _():
                    d_sd(ch + 2, (c4 + 2) % 4).start()

                d_xs(ch, c3, c4).wait()
                d_xd(ch, b2, c4).wait()
                d_g(ch, b2).wait()

                @pl.loop(0, CB2)
                def _edge(i):
                    xs_lo = comb[c3, i, pl.ds(0, 16)]
                    xs_hi = comb[c3, i, pl.ds(16, 16)]
                    m3 = (xs_lo + xdv[b2, i, pl.ds(0, 16)]
                          + g23v[b2, i, pl.ds(0, 16)])
                    v2 = (xs_hi + xdv[b2, i, pl.ds(16, 16)]
                          + g23v[b2, i, pl.ds(16, 16)])
                    ee3 = jnp.exp(_allsum16(_lr(m3) * att3))
                    ee2 = jnp.exp(_allsum16(_lr(v2) * att2h))
                    comb[c3, i, pl.ds(0, 16)] = xs_lo * ee3
                    comb[c3, i, pl.ds(16, 16)] = (
                        ee2 * (xs_hi * l0 + l1) + ee3 * l2)

                d_sc(c3, c4, b2).start(add=True)

    for chl in (NCH2 - 2, NCH2 - 1):
        d_sc(chl % 3, chl % 4, chl % 2).wait()

    plsc.subcore_barrier()
    pltpu.sync_copy(acc_sh.at[pl.ds(r0, RPT), :], out_hbm.at[cid, pl.ds(r0, RPT), :])


def _sc23_call(xf, g23, edge_index, att23, z32):
    f = pl.kernel(
        _sc23_body,
        out_type=jax.ShapeDtypeStruct((2, NNODE, 32), jnp.float32),
        mesh=_sc_mesh(),
        compiler_params=pltpu.CompilerParams(use_tc_tiling_on_sc=False),
        scratch_types=[
            pltpu.VMEM_SHARED((NNODE, 32), jnp.float32),
            pltpu.VMEM((4, 2, CB2), jnp.int32),
            pltpu.VMEM((3, CB2, 32), jnp.float32),
            pltpu.VMEM((2, CB2, 32), jnp.float32),
            pltpu.VMEM((2, CB2, 32), jnp.float32),
            pltpu.VMEM((32,), jnp.float32),
            pltpu.SemaphoreType.DMA((4,)),
            pltpu.SemaphoreType.DMA((3,)),
            pltpu.SemaphoreType.DMA((2,)),
            pltpu.SemaphoreType.DMA((2,)),
            pltpu.SemaphoreType.DMA((2,)),
        ],
    )
    return f(xf, g23, edge_index, att23, z32)


def kernel(x, edge_index, edge_attr, W1, att1, We1, b1, W2, att2, We2, b2,
           W3, att3, We3, b3):
    f32 = jnp.float32

    we23 = jnp.concatenate([We3, We2, jnp.zeros((16, 15), f32)], axis=1)

    xl1c = _matmul_ones(x, W1, 400)
    g1, g23 = _edge_proj(edge_attr, We1, we23)

    z32 = jnp.zeros((NNODE, 32), f32)
    z144 = jnp.zeros((NNODE, 144), f32)

    mp, cp = _sc0_call(edge_index, edge_attr, z32)

    p = _sc1_call(xl1c, g1, edge_index, att1, z144)

    att23 = jnp.concatenate([att3, att2, jnp.zeros((15,), f32)]).reshape(1, 32)
    w23 = jnp.concatenate([W3, W2, jnp.zeros((128, 15), f32)], axis=1)
    xf, es = _combine1(p[0], p[1], mp[0], mp[1], cp[0], cp[1],
                       xl1c, We1, att1.reshape(1, 128), b1.reshape(1, 128),
                       w23, we23, att23)

    q = _sc23_call(xf, g23, edge_index, att23.reshape(32), z32)

    nl, al = _combine2(q[0], q[1], xf, es, b2.reshape(1, 1), b3.reshape(1, 16))
    node_logits = nl[:, 0]
    action_logits = al

    node_sel = jax.random.categorical(jax.random.key(42), node_logits)
    node_lp = jax.nn.log_softmax(node_logits)[node_sel]
    alr = action_logits[node_sel, :]
    act_sel = jax.random.categorical(jax.random.key(43), alr)
    act_lp = jax.nn.log_softmax(alr)[act_sel]
    return (node_sel, act_sel, node_lp + act_lp)
